# baseline (speedup 1.0000x reference)
import jax
import jax.numpy as jnp
from jax import lax
from jax.experimental import pallas as pl
from jax.experimental.pallas import tpu as pltpu
from jax.experimental.pallas import tpu_sc as plsc

S = 1024
D = 128
NC = 2
NS = 16
NW = NC * NS
ROWS = S // NW
JC = 256
NCH = S // JC
CHUNK = JC * D


def _sc_body(x_hbm, rev_hbm, out_hbm, xbuf, ob0, ob1,
             isem0, isem1, osem0, osem1):
    wid = lax.axis_index("s") * NC + lax.axis_index("c")
    row0 = wid * ROWS

    def add_x(buf):
        def add_body(k, c2):
            off = pl.multiple_of(k * 128, 128)
            for u in range(8):
                sl = pl.ds(off + u * 16, 16)
                plsc.addupdate(buf.at[sl], xbuf[sl])
            return c2
        lax.fori_loop(0, CHUNK // 128, add_body, 0)

    for jc in range(NCH):
        col0 = jc * CHUNK
        pltpu.sync_copy(x_hbm.at[pl.ds(col0, CHUNK)], xbuf)

        def start_in(r, buf, sem, jc=jc):
            i = row0 + r
            start = pl.multiple_of((S - 1 - i + jc * JC) * D, D)
            pltpu.async_copy(rev_hbm.at[pl.ds(start, CHUNK)], buf, sem)

        start_in(0, ob0, isem0)
        start_in(1, ob1, isem1)

        def pair_body(k, carry, jc=jc, col0=col0, start_in=start_in):
            for b, buf, isem, osem in ((0, ob0, isem0, osem0),
                                       (1, ob1, isem1, osem1)):
                r = 2 * k + b
                pltpu.make_async_copy(
                    rev_hbm.at[pl.ds(0, CHUNK)], buf, isem).wait()
                add_x(buf)
                pltpu.async_copy(
                    buf, out_hbm.at[row0 + r, pl.ds(col0, CHUNK)], osem)

                @pl.when(k < ROWS // 2 - 1)
                def _(buf=buf, isem=isem, osem=osem, r=r):
                    pltpu.make_async_copy(
                        buf, out_hbm.at[0, pl.ds(0, CHUNK)], osem).wait()
                    start_in(r + 2, buf, isem)
            return carry

        lax.fori_loop(0, ROWS // 2, pair_body, 0)
        pltpu.make_async_copy(ob0, out_hbm.at[0, pl.ds(0, CHUNK)], osem0).wait()
        pltpu.make_async_copy(ob1, out_hbm.at[0, pl.ds(0, CHUNK)], osem1).wait()


def kernel(x, table):
    assert x.shape[1] == S and x.shape[2] == D
    rev = jnp.flip(table, axis=0)
    rev = jnp.pad(rev, ((0, 1), (0, 0))).reshape(2 * S * D)
    x_flat = x[0].reshape(S * D)

    mesh = plsc.VectorSubcoreMesh(
        core_axis_name="c", subcore_axis_name="s",
        num_cores=NC, num_subcores=NS,
    )
    sc_fn = pl.kernel(
        _sc_body,
        out_type=jax.ShapeDtypeStruct((S, S * D), jnp.float32),
        mesh=mesh,
        scratch_types=[
            pltpu.VMEM((CHUNK,), jnp.float32),
            pltpu.VMEM((CHUNK,), jnp.float32),
            pltpu.VMEM((CHUNK,), jnp.float32),
            pltpu.SemaphoreType.DMA,
            pltpu.SemaphoreType.DMA,
            pltpu.SemaphoreType.DMA,
            pltpu.SemaphoreType.DMA,
        ],
    )
    out = sc_fn(x_flat, rev)
    return out.reshape(S, S, D)


# speedup vs baseline: 1.0011x; 1.0011x over previous
import jax
import jax.numpy as jnp
from jax import lax
from jax.experimental import pallas as pl
from jax.experimental.pallas import tpu as pltpu
from jax.experimental.pallas import tpu_sc as plsc

S = 1024
D = 128
NC = 2
NS = 16
NW = NC * NS
ROWS = S // NW
JC = 256
NCH = S // JC
CHUNK = JC * D


def _sc_body(x_hbm, rev_hbm, out_hbm, xbuf, ob0, ob1,
             isem0, isem1, osem0, osem1):
    wid = lax.axis_index("s") * NC + lax.axis_index("c")
    row0 = wid * ROWS

    def add_x(buf):
        @plsc.parallel_loop(0, CHUNK, step=16, unroll=8)
        def add_body(i):
            sl = pl.ds(pl.multiple_of(i, 16), 16)
            plsc.addupdate(buf.at[sl], xbuf[sl])

    for jc in range(NCH):
        col0 = jc * CHUNK
        pltpu.sync_copy(x_hbm.at[pl.ds(col0, CHUNK)], xbuf)

        def start_in(r, buf, sem, jc=jc):
            i = row0 + r
            start = pl.multiple_of((S - 1 - i + jc * JC) * D, D)
            pltpu.async_copy(rev_hbm.at[pl.ds(start, CHUNK)], buf, sem)

        start_in(0, ob0, isem0)
        start_in(1, ob1, isem1)

        def pair_body(k, carry, jc=jc, col0=col0, start_in=start_in):
            for b, buf, isem, osem in ((0, ob0, isem0, osem0),
                                       (1, ob1, isem1, osem1)):
                r = 2 * k + b
                pltpu.make_async_copy(
                    rev_hbm.at[pl.ds(0, CHUNK)], buf, isem).wait()
                add_x(buf)
                pltpu.async_copy(
                    buf, out_hbm.at[row0 + r, pl.ds(col0, CHUNK)], osem)

                @pl.when(k < ROWS // 2 - 1)
                def _(buf=buf, isem=isem, osem=osem, r=r):
                    pltpu.make_async_copy(
                        buf, out_hbm.at[0, pl.ds(0, CHUNK)], osem).wait()
                    start_in(r + 2, buf, isem)
            return carry

        lax.fori_loop(0, ROWS // 2, pair_body, 0)
        pltpu.make_async_copy(ob0, out_hbm.at[0, pl.ds(0, CHUNK)], osem0).wait()
        pltpu.make_async_copy(ob1, out_hbm.at[0, pl.ds(0, CHUNK)], osem1).wait()


def kernel(x, table):
    assert x.shape[1] == S and x.shape[2] == D
    rev = jnp.flip(table, axis=0)
    rev = jnp.pad(rev, ((0, 1), (0, 0))).reshape(2 * S * D)
    x_flat = x[0].reshape(S * D)

    mesh = plsc.VectorSubcoreMesh(
        core_axis_name="c", subcore_axis_name="s",
        num_cores=NC, num_subcores=NS,
    )
    sc_fn = pl.kernel(
        _sc_body,
        out_type=jax.ShapeDtypeStruct((S, S * D), jnp.float32),
        mesh=mesh,
        scratch_types=[
            pltpu.VMEM((CHUNK,), jnp.float32),
            pltpu.VMEM((CHUNK,), jnp.float32),
            pltpu.VMEM((CHUNK,), jnp.float32),
            pltpu.SemaphoreType.DMA,
            pltpu.SemaphoreType.DMA,
            pltpu.SemaphoreType.DMA,
            pltpu.SemaphoreType.DMA,
        ],
    )
    out = sc_fn(x_flat, rev)
    return out.reshape(S, S, D)


# speedup vs baseline: 1.2109x; 1.2096x over previous
import jax
import jax.numpy as jnp
from jax import lax
from jax.experimental import pallas as pl
from jax.experimental.pallas import tpu as pltpu
from jax.experimental.pallas import tpu_sc as plsc

S = 1024
D = 128
NC = 2
NS = 16
NW = NC * NS
ROWS = S // NW
JC = 256
NCH = S // JC
CHUNK = JC * D


def _sc_body(x_hbm, rev_hbm, out_hbm, rev_sp, x_sp, xbuf, ob0, ob1,
             isem0, isem1, osem0, osem1):
    sid = lax.axis_index("s")
    wid = sid * NC + lax.axis_index("c")
    row0 = wid * ROWS

    @pl.when(sid == 0)
    def _():
        pltpu.sync_copy(rev_hbm, rev_sp)
        pltpu.sync_copy(x_hbm, x_sp)

    plsc.subcore_barrier()

    def add_x(buf):
        @plsc.parallel_loop(0, CHUNK, step=16, unroll=8)
        def add_body(i):
            sl = pl.ds(pl.multiple_of(i, 16), 16)
            plsc.addupdate(buf.at[sl], xbuf[sl])

    for jc in range(NCH):
        col0 = jc * CHUNK
        pltpu.sync_copy(x_sp.at[pl.ds(col0, CHUNK)], xbuf)

        def start_in(r, buf, sem, jc=jc):
            i = row0 + r
            start = pl.multiple_of((S - 1 - i + jc * JC) * D, D)
            pltpu.async_copy(rev_sp.at[pl.ds(start, CHUNK)], buf, sem)

        start_in(0, ob0, isem0)
        start_in(1, ob1, isem1)

        def pair_body(k, carry, jc=jc, col0=col0, start_in=start_in):
            for b, buf, isem, osem in ((0, ob0, isem0, osem0),
                                       (1, ob1, isem1, osem1)):
                r = 2 * k + b
                pltpu.make_async_copy(
                    rev_sp.at[pl.ds(0, CHUNK)], buf, isem).wait()
                add_x(buf)
                pltpu.async_copy(
                    buf, out_hbm.at[row0 + r, pl.ds(col0, CHUNK)], osem)

                @pl.when(k < ROWS // 2 - 1)
                def _(buf=buf, isem=isem, osem=osem, r=r):
                    pltpu.make_async_copy(
                        buf, out_hbm.at[0, pl.ds(0, CHUNK)], osem).wait()
                    start_in(r + 2, buf, isem)
            return carry

        lax.fori_loop(0, ROWS // 2, pair_body, 0)
        pltpu.make_async_copy(ob0, out_hbm.at[0, pl.ds(0, CHUNK)], osem0).wait()
        pltpu.make_async_copy(ob1, out_hbm.at[0, pl.ds(0, CHUNK)], osem1).wait()


def kernel(x, table):
    assert x.shape[1] == S and x.shape[2] == D
    rev = jnp.flip(table, axis=0)
    rev = jnp.pad(rev, ((0, 1), (0, 0))).reshape(2 * S * D)
    x_flat = x[0].reshape(S * D)

    mesh = plsc.VectorSubcoreMesh(
        core_axis_name="c", subcore_axis_name="s",
        num_cores=NC, num_subcores=NS,
    )
    sc_fn = pl.kernel(
        _sc_body,
        out_type=jax.ShapeDtypeStruct((S, S * D), jnp.float32),
        mesh=mesh,
        scratch_types=[
            pltpu.MemorySpace.VMEM_SHARED((2 * S * D,), jnp.float32),
            pltpu.MemorySpace.VMEM_SHARED((S * D,), jnp.float32),
            pltpu.VMEM((CHUNK,), jnp.float32),
            pltpu.VMEM((CHUNK,), jnp.float32),
            pltpu.VMEM((CHUNK,), jnp.float32),
            pltpu.SemaphoreType.DMA,
            pltpu.SemaphoreType.DMA,
            pltpu.SemaphoreType.DMA,
            pltpu.SemaphoreType.DMA,
        ],
    )
    out = sc_fn(x_flat, rev)
    return out.reshape(S, S, D)


# speedup vs baseline: 1.4174x; 1.1706x over previous
import jax
import jax.numpy as jnp
from jax import lax
from jax.experimental import pallas as pl
from jax.experimental.pallas import tpu as pltpu
from jax.experimental.pallas import tpu_sc as plsc

S = 1024
D = 128
NC = 2
NS = 16
NW = NC * NS
ROWS = S // NW
JC = 128
NCH = S // JC
CHUNK = JC * D
WIN = (JC + ROWS - 1) * D


def _sc_body(x_hbm, rev_hbm, out_hbm, wbuf, xbuf, ob0, ob1, osem0, osem1):
    wid = lax.axis_index("s") * NC + lax.axis_index("c")
    row0 = wid * ROWS

    def build(buf, woff):
        @plsc.parallel_loop(0, CHUNK, step=16, unroll=8)
        def body(kk):
            sl = pl.ds(pl.multiple_of(kk, 16), 16)
            slw = pl.ds(pl.multiple_of(woff + kk, 16), 16)
            buf[sl] = wbuf[slw] + xbuf[sl]

    for jc in range(NCH):
        col0 = jc * CHUNK
        pltpu.sync_copy(x_hbm.at[pl.ds(col0, CHUNK)], xbuf)
        wstart = pl.multiple_of((S - 1 - (row0 + ROWS - 1) + jc * JC) * D, D)
        pltpu.sync_copy(rev_hbm.at[pl.ds(wstart, WIN)], wbuf)

        def pair_body(k, carry, col0=col0):
            for b, buf, osem in ((0, ob0, osem0), (1, ob1, osem1)):
                r = 2 * k + b

                @pl.when(k > 0)
                def _(buf=buf, osem=osem):
                    pltpu.make_async_copy(
                        buf, out_hbm.at[0, pl.ds(0, CHUNK)], osem).wait()

                build(buf, (ROWS - 1 - r) * D)
                pltpu.async_copy(
                    buf, out_hbm.at[row0 + r, pl.ds(col0, CHUNK)], osem)
            return carry

        lax.fori_loop(0, ROWS // 2, pair_body, 0)
        pltpu.make_async_copy(ob0, out_hbm.at[0, pl.ds(0, CHUNK)], osem0).wait()
        pltpu.make_async_copy(ob1, out_hbm.at[0, pl.ds(0, CHUNK)], osem1).wait()


def kernel(x, table):
    assert x.shape[1] == S and x.shape[2] == D
    rev = jnp.flip(table, axis=0)
    rev = jnp.pad(rev, ((0, 1), (0, 0))).reshape(2 * S * D)
    x_flat = x[0].reshape(S * D)

    mesh = plsc.VectorSubcoreMesh(
        core_axis_name="c", subcore_axis_name="s",
        num_cores=NC, num_subcores=NS,
    )
    sc_fn = pl.kernel(
        _sc_body,
        out_type=jax.ShapeDtypeStruct((S, S * D), jnp.float32),
        mesh=mesh,
        scratch_types=[
            pltpu.VMEM((WIN,), jnp.float32),
            pltpu.VMEM((CHUNK,), jnp.float32),
            pltpu.VMEM((CHUNK,), jnp.float32),
            pltpu.VMEM((CHUNK,), jnp.float32),
            pltpu.SemaphoreType.DMA,
            pltpu.SemaphoreType.DMA,
        ],
    )
    out = sc_fn(x_flat, rev)
    return out.reshape(S, S, D)


# speedup vs baseline: 1.4175x; 1.0000x over previous
import jax
import jax.numpy as jnp
from jax import lax
from jax.experimental import pallas as pl
from jax.experimental.pallas import tpu as pltpu
from jax.experimental.pallas import tpu_sc as plsc

S = 1024
D = 128
NC = 2
NS = 16
NW = NC * NS
ROWS = S // NW
JC = 128
NCH = S // JC
CHUNK = JC * D
WIN = (JC + ROWS - 1) * D


def _sc_body(x_hbm, rev_hbm, out_hbm, wbuf, xbuf, ob0, ob1, osem0, osem1):
    wid = lax.axis_index("s") * NC + lax.axis_index("c")
    row0 = wid * ROWS

    def build(buf, woff):
        @plsc.parallel_loop(0, CHUNK, step=16, unroll=16)
        def body(kk):
            sl = pl.ds(pl.multiple_of(kk, 16), 16)
            slw = pl.ds(pl.multiple_of(woff + kk, 16), 16)
            buf[sl] = wbuf[slw] + xbuf[sl]

    for jc in range(NCH):
        col0 = jc * CHUNK
        pltpu.sync_copy(x_hbm.at[pl.ds(col0, CHUNK)], xbuf)
        wstart = pl.multiple_of((S - 1 - (row0 + ROWS - 1) + jc * JC) * D, D)
        pltpu.sync_copy(rev_hbm.at[pl.ds(wstart, WIN)], wbuf)

        def pair_body(k, carry, col0=col0):
            for b, buf, osem in ((0, ob0, osem0), (1, ob1, osem1)):
                r = 2 * k + b

                @pl.when(k > 0)
                def _(buf=buf, osem=osem):
                    pltpu.make_async_copy(
                        buf, out_hbm.at[0, pl.ds(0, CHUNK)], osem).wait()

                build(buf, (ROWS - 1 - r) * D)
                pltpu.async_copy(
                    buf, out_hbm.at[row0 + r, pl.ds(col0, CHUNK)], osem)
            return carry

        lax.fori_loop(0, ROWS // 2, pair_body, 0)
        pltpu.make_async_copy(ob0, out_hbm.at[0, pl.ds(0, CHUNK)], osem0).wait()
        pltpu.make_async_copy(ob1, out_hbm.at[0, pl.ds(0, CHUNK)], osem1).wait()


def kernel(x, table):
    assert x.shape[1] == S and x.shape[2] == D
    rev = jnp.flip(table, axis=0)
    rev = jnp.pad(rev, ((0, 1), (0, 0))).reshape(2 * S * D)
    x_flat = x[0].reshape(S * D)

    mesh = plsc.VectorSubcoreMesh(
        core_axis_name="c", subcore_axis_name="s",
        num_cores=NC, num_subcores=NS,
    )
    sc_fn = pl.kernel(
        _sc_body,
        out_type=jax.ShapeDtypeStruct((S, S * D), jnp.float32),
        mesh=mesh,
        scratch_types=[
            pltpu.VMEM((WIN,), jnp.float32),
            pltpu.VMEM((CHUNK,), jnp.float32),
            pltpu.VMEM((CHUNK,), jnp.float32),
            pltpu.VMEM((CHUNK,), jnp.float32),
            pltpu.SemaphoreType.DMA,
            pltpu.SemaphoreType.DMA,
        ],
    )
    out = sc_fn(x_flat, rev)
    return out.reshape(S, S, D)


# speedup vs baseline: 5.9642x; 4.2076x over previous
import jax
import jax.numpy as jnp
from jax.experimental import pallas as pl
from jax.experimental.pallas import tpu as pltpu

_BI = 16


def _row_block_kernel(x_ref, rev_ref, o_ref):
    i0 = pl.program_id(0) * _BI
    seq = x_ref.shape[0]
    for di in range(_BI):
        start = (seq - 1) - (i0 + di)
        o_ref[di] = x_ref[...] + rev_ref[pl.ds(start, seq), :]


def kernel(x, table):
    seq = x.shape[1]
    d = x.shape[2]
    maxlen = (table.shape[0] + 1) // 2
    assert maxlen == seq
    rev = jnp.flip(table, axis=0)
    rev = jnp.pad(rev, ((0, 1), (0, 0)))
    x2 = x[0]

    out = pl.pallas_call(
        _row_block_kernel,
        grid=(seq // _BI,),
        in_specs=[
            pl.BlockSpec((seq, d), lambda i: (0, 0)),
            pl.BlockSpec((2 * seq, d), lambda i: (0, 0)),
        ],
        out_specs=pl.BlockSpec((_BI, seq, d), lambda i: (i, 0, 0)),
        out_shape=jax.ShapeDtypeStruct((seq, seq, d), x.dtype),
    )(x2, rev)
    return out


# speedup vs baseline: 6.1295x; 1.0277x over previous
import jax
import jax.numpy as jnp
from jax.experimental import pallas as pl

_BI = 16


def _row_block_kernel(x_ref, rev_ref, o_ref):
    i0 = pl.program_id(0) * _BI
    seq = x_ref.shape[0]
    for di in range(_BI):
        start = (seq - 1) - (i0 + di)
        o_ref[di] = x_ref[...] + rev_ref[pl.ds(start, seq), :]


def kernel(x, table):
    seq = x.shape[1]
    d = x.shape[2]
    maxlen = (table.shape[0] + 1) // 2
    assert maxlen == seq
    rev = jnp.flip(table, axis=0)
    rev = jnp.pad(rev, ((0, 1), (0, 0)))
    x2 = x[0]

    out = pl.pallas_call(
        _row_block_kernel,
        grid=(seq // _BI,),
        in_specs=[
            pl.BlockSpec((seq, d), lambda i: (0, 0)),
            pl.BlockSpec((2 * seq, d), lambda i: (0, 0)),
        ],
        out_specs=pl.BlockSpec((_BI, seq, d), lambda i: (i, 0, 0)),
        out_shape=jax.ShapeDtypeStruct((seq, seq, d), x.dtype),
    )(x2, rev)
    return out
